# Initial kernel scaffold; baseline (speedup 1.0000x reference)
import jax
import jax.numpy as jnp
from jax.experimental import pallas as pl


def kernel(x_real, x_imag, cb_real, cb_imag):
    raise NotImplementedError("write your pallas kernel here")


# speedup vs baseline: 1.1145x; 1.1145x over previous
import functools

import jax
import jax.numpy as jnp
from jax.experimental import pallas as pl

B, N, D = 16, 576, 256
K = 8192
M = B * N

MBLK = 256
KBLK = 2048


def _vq_kernel(xr_ref, xi_ref, cr_ref, ci_ref, out_ref):
    xr = xr_ref[...]
    xi = xi_ref[...]
    xd = xr - xi

    nchunks = K // KBLK

    def body(c, carry):
        run_max, run_idx = carry
        k0 = c * KBLK
        cr = cr_ref[pl.ds(k0, KBLK), :]
        ci = ci_ref[pl.ds(k0, KBLK), :]
        dot = functools.partial(
            jax.lax.dot_general,
            dimension_numbers=(((1,), (1,)), ((), ())),
            preferred_element_type=jnp.float32,
        )
        p1 = dot(xd, cr)
        p2 = dot(xi, cr + ci)
        p3 = dot(xr, ci - cr)
        re = p1 + p2
        im = p1 + p3
        are = jnp.abs(re)
        aim = jnp.abs(im)
        mx = jnp.maximum(are, aim)
        mn = jnp.minimum(are, aim)
        q = mn / mx
        sc = mx * jnp.sqrt(1.0 + q * q)
        sc = jnp.where(jnp.isnan(sc), mn, sc)

        cmax = jnp.max(sc, axis=1, keepdims=True)
        iota = jax.lax.broadcasted_iota(jnp.int32, sc.shape, 1)
        carg = jnp.min(
            jnp.where(sc == cmax, iota, K), axis=1, keepdims=True
        ) + k0

        better = cmax > run_max
        return (
            jnp.where(better, cmax, run_max),
            jnp.where(better, carg, run_idx),
        )

    init = (
        jnp.full((MBLK, 1), -jnp.inf, dtype=jnp.float32),
        jnp.zeros((MBLK, 1), dtype=jnp.int32),
    )
    _, idx = jax.lax.fori_loop(0, nchunks, body, init)
    out_ref[...] = idx.reshape(1, 1, MBLK)


def kernel(x_real, x_imag, cb_real, cb_imag):
    xr = x_real.reshape(M, D)
    xi = x_imag.reshape(M, D)

    nm = M // MBLK
    out = pl.pallas_call(
        _vq_kernel,
        grid=(nm,),
        in_specs=[
            pl.BlockSpec((MBLK, D), lambda m: (m, 0)),
            pl.BlockSpec((MBLK, D), lambda m: (m, 0)),
            pl.BlockSpec((K, D), lambda m: (0, 0)),
            pl.BlockSpec((K, D), lambda m: (0, 0)),
        ],
        out_specs=pl.BlockSpec((1, 1, MBLK), lambda m: (m, 0, 0)),
        out_shape=jax.ShapeDtypeStruct((nm, 1, MBLK), jnp.int32),
    )(xr, xi, cb_real, cb_imag)
    return out.reshape(B, N)


# speedup vs baseline: 1.5008x; 1.3465x over previous
import functools

import jax
import jax.numpy as jnp
from jax.experimental import pallas as pl
from jax.experimental.pallas import tpu as pltpu

B, N, D = 16, 576, 256
K = 8192
M = B * N

MBLK = 256
KBLK = 2048
GUARD = 1e-5

_dot = functools.partial(
    jax.lax.dot_general,
    dimension_numbers=(((1,), (1,)), ((), ())),
    preferred_element_type=jnp.float32,
)


def _vq_kernel(xr_ref, xi_ref, cr_ref, ci_ref, out_ref, u_ref):
    xr = xr_ref[...]
    xi = xi_ref[...]
    xd = xr - xi

    nchunks = K // KBLK

    def re_im(k0):
        cr = cr_ref[pl.ds(k0, KBLK), :]
        ci = ci_ref[pl.ds(k0, KBLK), :]
        p1 = _dot(xd, cr)
        p2 = _dot(xi, cr + ci)
        p3 = _dot(xr, ci - cr)
        return p1 + p2, p1 + p3

    def body_a(c, umax):
        k0 = c * KBLK
        re, im = re_im(k0)
        u = re * re + im * im
        u_ref[:, pl.ds(k0, KBLK)] = u
        return jnp.maximum(umax, jnp.max(u, axis=1, keepdims=True))

    umax = jax.lax.fori_loop(
        0, nchunks, body_a, jnp.full((MBLK, 1), -jnp.inf, jnp.float32)
    )

    thresh = umax * (1.0 - GUARD)

    def body_b(c, carry):
        cnt, argf = carry
        k0 = c * KBLK
        u = u_ref[:, pl.ds(k0, KBLK)]
        iota = jax.lax.broadcasted_iota(jnp.int32, u.shape, 1) + k0
        argc = jnp.min(
            jnp.where(u == umax, iota, K), axis=1, keepdims=True
        )
        near = (u >= thresh).astype(jnp.int32)
        return (
            cnt + jnp.sum(near, axis=1, keepdims=True),
            jnp.minimum(argf, argc),
        )

    cnt, argf = jax.lax.fori_loop(
        0,
        nchunks,
        body_b,
        (
            jnp.zeros((MBLK, 1), jnp.int32),
            jnp.full((MBLK, 1), K, jnp.int32),
        ),
    )
    out_ref[...] = argf.reshape(1, 1, MBLK)

    ambiguous = jnp.sum((cnt != 1).astype(jnp.int32)) > 0

    @pl.when(ambiguous)
    def _exact_path():
        def body(c, carry):
            run_max, run_idx = carry
            k0 = c * KBLK
            re, im = re_im(k0)
            are = jnp.abs(re)
            aim = jnp.abs(im)
            mx = jnp.maximum(are, aim)
            mn = jnp.minimum(are, aim)
            q = mn / mx
            sc = mx * jnp.sqrt(1.0 + q * q)
            sc = jnp.where(jnp.isnan(sc), mn, sc)

            cmax = jnp.max(sc, axis=1, keepdims=True)
            iota = jax.lax.broadcasted_iota(jnp.int32, sc.shape, 1)
            carg = jnp.min(
                jnp.where(sc == cmax, iota, K), axis=1, keepdims=True
            ) + k0

            better = cmax > run_max
            return (
                jnp.where(better, cmax, run_max),
                jnp.where(better, carg, run_idx),
            )

        init = (
            jnp.full((MBLK, 1), -jnp.inf, dtype=jnp.float32),
            jnp.zeros((MBLK, 1), dtype=jnp.int32),
        )
        _, idx = jax.lax.fori_loop(0, nchunks, body, init)
        out_ref[...] = idx.reshape(1, 1, MBLK)


def kernel(x_real, x_imag, cb_real, cb_imag):
    xr = x_real.reshape(M, D)
    xi = x_imag.reshape(M, D)

    nm = M // MBLK
    out = pl.pallas_call(
        _vq_kernel,
        grid=(nm,),
        in_specs=[
            pl.BlockSpec((MBLK, D), lambda m: (m, 0)),
            pl.BlockSpec((MBLK, D), lambda m: (m, 0)),
            pl.BlockSpec((K, D), lambda m: (0, 0)),
            pl.BlockSpec((K, D), lambda m: (0, 0)),
        ],
        out_specs=pl.BlockSpec((1, 1, MBLK), lambda m: (m, 0, 0)),
        out_shape=jax.ShapeDtypeStruct((nm, 1, MBLK), jnp.int32),
        scratch_shapes=[pltpu.VMEM((MBLK, K), jnp.float32)],
    )(xr, xi, cb_real, cb_imag)
    return out.reshape(B, N)


# speedup vs baseline: 1.6970x; 1.1308x over previous
import functools

import jax
import jax.numpy as jnp
from jax.experimental import pallas as pl

B, N, D = 16, 576, 256
K = 8192
M = B * N

MBLK = 256
KBLK = 2048
GUARD = 1e-5

_dot = functools.partial(
    jax.lax.dot_general,
    dimension_numbers=(((1,), (1,)), ((), ())),
    preferred_element_type=jnp.float32,
)


def _vq_kernel(xr_ref, xi_ref, cr_ref, ci_ref, out_ref):
    xr = xr_ref[...]
    xi = xi_ref[...]
    xd = xr - xi

    nchunks = K // KBLK
    nslices = KBLK // 128

    def re_im(k0):
        cr = cr_ref[pl.ds(k0, KBLK), :]
        ci = ci_ref[pl.ds(k0, KBLK), :]
        p1 = _dot(xd, cr)
        p2 = _dot(xi, cr + ci)
        p3 = _dot(xr, ci - cr)
        return p1 + p2, p1 + p3

    def body_a(c, carry):
        m1, m2, lslot = carry
        k0 = c * KBLK
        re, im = re_im(k0)
        u = re * re + im * im
        for j in range(nslices):
            s = u[:, j * 128:(j + 1) * 128]
            slot = c * nslices + j
            gt = s > m1
            m2 = jnp.maximum(m2, jnp.minimum(m1, s))
            m1 = jnp.maximum(m1, s)
            lslot = jnp.where(gt, slot, lslot)
        return m1, m2, lslot

    m1, m2, lslot = jax.lax.fori_loop(
        0,
        nchunks,
        body_a,
        (
            jnp.full((MBLK, 128), -jnp.inf, jnp.float32),
            jnp.full((MBLK, 128), -jnp.inf, jnp.float32),
            jnp.zeros((MBLK, 128), jnp.int32),
        ),
    )

    umax = jnp.max(m1, axis=1, keepdims=True)
    thresh = umax * (1.0 - GUARD)
    cnt = jnp.sum((m1 >= thresh).astype(jnp.int32), axis=1, keepdims=True)
    cnt = cnt + jnp.sum(
        (m2 >= thresh).astype(jnp.int32), axis=1, keepdims=True
    )
    lane = jax.lax.broadcasted_iota(jnp.int32, (MBLK, 128), 1)
    kglob = lslot * 128 + lane
    argf = jnp.min(jnp.where(m1 == umax, kglob, K), axis=1, keepdims=True)
    out_ref[...] = argf.reshape(1, 1, MBLK)

    ambiguous = jnp.sum((cnt != 1).astype(jnp.int32)) > 0

    @pl.when(ambiguous)
    def _exact_path():
        def body(c, carry):
            run_max, run_idx = carry
            k0 = c * KBLK
            re, im = re_im(k0)
            are = jnp.abs(re)
            aim = jnp.abs(im)
            mx = jnp.maximum(are, aim)
            mn = jnp.minimum(are, aim)
            q = mn / mx
            sc = mx * jnp.sqrt(1.0 + q * q)
            sc = jnp.where(jnp.isnan(sc), mn, sc)

            cmax = jnp.max(sc, axis=1, keepdims=True)
            iota = jax.lax.broadcasted_iota(jnp.int32, sc.shape, 1)
            carg = jnp.min(
                jnp.where(sc == cmax, iota, K), axis=1, keepdims=True
            ) + k0

            better = cmax > run_max
            return (
                jnp.where(better, cmax, run_max),
                jnp.where(better, carg, run_idx),
            )

        init = (
            jnp.full((MBLK, 1), -jnp.inf, dtype=jnp.float32),
            jnp.zeros((MBLK, 1), dtype=jnp.int32),
        )
        _, idx = jax.lax.fori_loop(0, nchunks, body, init)
        out_ref[...] = idx.reshape(1, 1, MBLK)


def kernel(x_real, x_imag, cb_real, cb_imag):
    xr = x_real.reshape(M, D)
    xi = x_imag.reshape(M, D)

    nm = M // MBLK
    out = pl.pallas_call(
        _vq_kernel,
        grid=(nm,),
        in_specs=[
            pl.BlockSpec((MBLK, D), lambda m: (m, 0)),
            pl.BlockSpec((MBLK, D), lambda m: (m, 0)),
            pl.BlockSpec((K, D), lambda m: (0, 0)),
            pl.BlockSpec((K, D), lambda m: (0, 0)),
        ],
        out_specs=pl.BlockSpec((1, 1, MBLK), lambda m: (m, 0, 0)),
        out_shape=jax.ShapeDtypeStruct((nm, 1, MBLK), jnp.int32),
    )(xr, xi, cb_real, cb_imag)
    return out.reshape(B, N)


# speedup vs baseline: 1.7171x; 1.0118x over previous
import functools

import jax
import jax.numpy as jnp
from jax.experimental import pallas as pl

B, N, D = 16, 576, 256
K = 8192
M = B * N

MBLK = 256
KBLK = 2048
GUARD = 1e-5

_dot = functools.partial(
    jax.lax.dot_general,
    dimension_numbers=(((1,), (1,)), ((), ())),
    preferred_element_type=jnp.float32,
)


def _vq_kernel(xr_ref, xi_ref, cr_ref, cpc_ref, cmc_ref, out_ref):
    xr = xr_ref[...]
    xi = xi_ref[...]
    xd = xr - xi

    nchunks = K // KBLK
    nslices = KBLK // 128

    def re_im(k0):
        cr = cr_ref[pl.ds(k0, KBLK), :]
        cpc = cpc_ref[pl.ds(k0, KBLK), :]
        cmc = cmc_ref[pl.ds(k0, KBLK), :]
        p1 = _dot(xd, cr)
        p2 = _dot(xi, cpc)
        p3 = _dot(xr, cmc)
        return p1 + p2, p1 + p3

    def body_a(c, carry):
        m1, m2, lslot = carry
        k0 = c * KBLK
        re, im = re_im(k0)
        u = re * re + im * im
        for j in range(nslices):
            s = u[:, j * 128:(j + 1) * 128]
            slot = c * nslices + j
            gt = s > m1
            m2 = jnp.maximum(m2, jnp.minimum(m1, s))
            m1 = jnp.maximum(m1, s)
            lslot = jnp.where(gt, slot, lslot)
        return m1, m2, lslot

    m1, m2, lslot = jax.lax.fori_loop(
        0,
        nchunks,
        body_a,
        (
            jnp.full((MBLK, 128), -jnp.inf, jnp.float32),
            jnp.full((MBLK, 128), -jnp.inf, jnp.float32),
            jnp.zeros((MBLK, 128), jnp.int32),
        ),
    )

    umax = jnp.max(m1, axis=1, keepdims=True)
    thresh = umax * (1.0 - GUARD)
    cnt = jnp.sum((m1 >= thresh).astype(jnp.int32), axis=1, keepdims=True)
    cnt = cnt + jnp.sum(
        (m2 >= thresh).astype(jnp.int32), axis=1, keepdims=True
    )
    lane = jax.lax.broadcasted_iota(jnp.int32, (MBLK, 128), 1)
    kglob = lslot * 128 + lane
    argf = jnp.min(jnp.where(m1 == umax, kglob, K), axis=1, keepdims=True)
    out_ref[...] = argf.reshape(1, 1, MBLK)

    ambiguous = jnp.sum((cnt != 1).astype(jnp.int32)) > 0

    @pl.when(ambiguous)
    def _exact_path():
        def body(c, carry):
            run_max, run_idx = carry
            k0 = c * KBLK
            re, im = re_im(k0)
            are = jnp.abs(re)
            aim = jnp.abs(im)
            mx = jnp.maximum(are, aim)
            mn = jnp.minimum(are, aim)
            q = mn / mx
            sc = mx * jnp.sqrt(1.0 + q * q)
            sc = jnp.where(jnp.isnan(sc), mn, sc)

            cmax = jnp.max(sc, axis=1, keepdims=True)
            iota = jax.lax.broadcasted_iota(jnp.int32, sc.shape, 1)
            carg = jnp.min(
                jnp.where(sc == cmax, iota, K), axis=1, keepdims=True
            ) + k0

            better = cmax > run_max
            return (
                jnp.where(better, cmax, run_max),
                jnp.where(better, carg, run_idx),
            )

        init = (
            jnp.full((MBLK, 1), -jnp.inf, dtype=jnp.float32),
            jnp.zeros((MBLK, 1), dtype=jnp.int32),
        )
        _, idx = jax.lax.fori_loop(0, nchunks, body, init)
        out_ref[...] = idx.reshape(1, 1, MBLK)


def kernel(x_real, x_imag, cb_real, cb_imag):
    xr = x_real.reshape(M, D)
    xi = x_imag.reshape(M, D)
    cpc = cb_real + cb_imag
    cmc = cb_imag - cb_real

    nm = M // MBLK
    out = pl.pallas_call(
        _vq_kernel,
        grid=(nm,),
        in_specs=[
            pl.BlockSpec((MBLK, D), lambda m: (m, 0)),
            pl.BlockSpec((MBLK, D), lambda m: (m, 0)),
            pl.BlockSpec((K, D), lambda m: (0, 0)),
            pl.BlockSpec((K, D), lambda m: (0, 0)),
            pl.BlockSpec((K, D), lambda m: (0, 0)),
        ],
        out_specs=pl.BlockSpec((1, 1, MBLK), lambda m: (m, 0, 0)),
        out_shape=jax.ShapeDtypeStruct((nm, 1, MBLK), jnp.int32),
    )(xr, xi, cb_real, cpc, cmc)
    return out.reshape(B, N)


# speedup vs baseline: 1.7739x; 1.0331x over previous
import functools

import jax
import jax.numpy as jnp
from jax.experimental import pallas as pl

B, N, D = 16, 576, 256
K = 8192
M = B * N

MBLK = 256
KBLK = 4096
GUARD = 1e-5

_dot = functools.partial(
    jax.lax.dot_general,
    dimension_numbers=(((1,), (1,)), ((), ())),
    preferred_element_type=jnp.float32,
)


def _vq_kernel(xr_ref, xi_ref, cr_ref, cpc_ref, cmc_ref, out_ref):
    xr = xr_ref[...]
    xi = xi_ref[...]
    xd = xr - xi

    nchunks = K // KBLK
    nslices = KBLK // 128

    def re_im(k0):
        cr = cr_ref[pl.ds(k0, KBLK), :]
        cpc = cpc_ref[pl.ds(k0, KBLK), :]
        cmc = cmc_ref[pl.ds(k0, KBLK), :]
        p1 = _dot(xd, cr)
        p2 = _dot(xi, cpc)
        p3 = _dot(xr, cmc)
        return p1 + p2, p1 + p3

    def body_a(c, carry):
        m1, m2, lslot = carry
        k0 = c * KBLK
        re, im = re_im(k0)
        u = re * re + im * im
        for j in range(nslices):
            s = u[:, j * 128:(j + 1) * 128]
            slot = c * nslices + j
            gt = s > m1
            m2 = jnp.maximum(m2, jnp.minimum(m1, s))
            m1 = jnp.maximum(m1, s)
            lslot = jnp.where(gt, slot, lslot)
        return m1, m2, lslot

    m1, m2, lslot = jax.lax.fori_loop(
        0,
        nchunks,
        body_a,
        (
            jnp.full((MBLK, 128), -jnp.inf, jnp.float32),
            jnp.full((MBLK, 128), -jnp.inf, jnp.float32),
            jnp.zeros((MBLK, 128), jnp.int32),
        ),
    )

    umax = jnp.max(m1, axis=1, keepdims=True)
    thresh = umax * (1.0 - GUARD)
    cnt = jnp.sum((m1 >= thresh).astype(jnp.int32), axis=1, keepdims=True)
    cnt = cnt + jnp.sum(
        (m2 >= thresh).astype(jnp.int32), axis=1, keepdims=True
    )
    lane = jax.lax.broadcasted_iota(jnp.int32, (MBLK, 128), 1)
    kglob = lslot * 128 + lane
    argf = jnp.min(jnp.where(m1 == umax, kglob, K), axis=1, keepdims=True)
    out_ref[...] = argf.reshape(1, 1, MBLK)

    ambiguous = jnp.sum((cnt != 1).astype(jnp.int32)) > 0

    @pl.when(ambiguous)
    def _exact_path():
        def body(c, carry):
            run_max, run_idx = carry
            k0 = c * KBLK
            re, im = re_im(k0)
            are = jnp.abs(re)
            aim = jnp.abs(im)
            mx = jnp.maximum(are, aim)
            mn = jnp.minimum(are, aim)
            q = mn / mx
            sc = mx * jnp.sqrt(1.0 + q * q)
            sc = jnp.where(jnp.isnan(sc), mn, sc)

            cmax = jnp.max(sc, axis=1, keepdims=True)
            iota = jax.lax.broadcasted_iota(jnp.int32, sc.shape, 1)
            carg = jnp.min(
                jnp.where(sc == cmax, iota, K), axis=1, keepdims=True
            ) + k0

            better = cmax > run_max
            return (
                jnp.where(better, cmax, run_max),
                jnp.where(better, carg, run_idx),
            )

        init = (
            jnp.full((MBLK, 1), -jnp.inf, dtype=jnp.float32),
            jnp.zeros((MBLK, 1), dtype=jnp.int32),
        )
        _, idx = jax.lax.fori_loop(0, nchunks, body, init)
        out_ref[...] = idx.reshape(1, 1, MBLK)


def kernel(x_real, x_imag, cb_real, cb_imag):
    xr = x_real.reshape(M, D)
    xi = x_imag.reshape(M, D)
    cpc = cb_real + cb_imag
    cmc = cb_imag - cb_real

    nm = M // MBLK
    out = pl.pallas_call(
        _vq_kernel,
        grid=(nm,),
        in_specs=[
            pl.BlockSpec((MBLK, D), lambda m: (m, 0)),
            pl.BlockSpec((MBLK, D), lambda m: (m, 0)),
            pl.BlockSpec((K, D), lambda m: (0, 0)),
            pl.BlockSpec((K, D), lambda m: (0, 0)),
            pl.BlockSpec((K, D), lambda m: (0, 0)),
        ],
        out_specs=pl.BlockSpec((1, 1, MBLK), lambda m: (m, 0, 0)),
        out_shape=jax.ShapeDtypeStruct((nm, 1, MBLK), jnp.int32),
    )(xr, xi, cb_real, cpc, cmc)
    return out.reshape(B, N)


# speedup vs baseline: 1.8548x; 1.0456x over previous
import functools

import jax
import jax.numpy as jnp
from jax.experimental import pallas as pl
from jax.experimental.pallas import tpu as pltpu

B, N, D = 16, 576, 256
K = 8192
M = B * N

MBLK = 256
KBLK = 4096
GUARD = 1e-5

_dot = functools.partial(
    jax.lax.dot_general,
    dimension_numbers=(((1,), (1,)), ((), ())),
    preferred_element_type=jnp.float32,
)


def _vq_kernel(xr_ref, xi_ref, cr_ref, ci_ref, out_ref, cpc_ref, cmc_ref):
    @pl.when(pl.program_id(0) == 0)
    def _prep():
        cr_all = cr_ref[...]
        ci_all = ci_ref[...]
        cpc_ref[...] = cr_all + ci_all
        cmc_ref[...] = ci_all - cr_all

    xr = xr_ref[...]
    xi = xi_ref[...]
    xd = xr - xi

    nchunks = K // KBLK
    nslices = KBLK // 128

    def re_im(k0):
        cr = cr_ref[pl.ds(k0, KBLK), :]
        cpc = cpc_ref[pl.ds(k0, KBLK), :]
        cmc = cmc_ref[pl.ds(k0, KBLK), :]
        p1 = _dot(xd, cr)
        p2 = _dot(xi, cpc)
        p3 = _dot(xr, cmc)
        return p1 + p2, p1 + p3

    def body_a(c, carry):
        m1, m2, lslot = carry
        k0 = c * KBLK
        re, im = re_im(k0)
        u = re * re + im * im
        for j in range(nslices):
            s = u[:, j * 128:(j + 1) * 128]
            slot = c * nslices + j
            gt = s > m1
            m2 = jnp.maximum(m2, jnp.minimum(m1, s))
            m1 = jnp.maximum(m1, s)
            lslot = jnp.where(gt, slot, lslot)
        return m1, m2, lslot

    m1, m2, lslot = jax.lax.fori_loop(
        0,
        nchunks,
        body_a,
        (
            jnp.full((MBLK, 128), -jnp.inf, jnp.float32),
            jnp.full((MBLK, 128), -jnp.inf, jnp.float32),
            jnp.zeros((MBLK, 128), jnp.int32),
        ),
    )

    umax = jnp.max(m1, axis=1, keepdims=True)
    thresh = umax * (1.0 - GUARD)
    cnt = jnp.sum((m1 >= thresh).astype(jnp.int32), axis=1, keepdims=True)
    cnt = cnt + jnp.sum(
        (m2 >= thresh).astype(jnp.int32), axis=1, keepdims=True
    )
    lane = jax.lax.broadcasted_iota(jnp.int32, (MBLK, 128), 1)
    kglob = lslot * 128 + lane
    argf = jnp.min(jnp.where(m1 == umax, kglob, K), axis=1, keepdims=True)
    out_ref[...] = argf.reshape(1, 1, MBLK)

    ambiguous = jnp.sum((cnt != 1).astype(jnp.int32)) > 0

    @pl.when(ambiguous)
    def _exact_path():
        def body(c, carry):
            run_max, run_idx = carry
            k0 = c * KBLK
            re, im = re_im(k0)
            are = jnp.abs(re)
            aim = jnp.abs(im)
            mx = jnp.maximum(are, aim)
            mn = jnp.minimum(are, aim)
            q = mn / mx
            sc = mx * jnp.sqrt(1.0 + q * q)
            sc = jnp.where(jnp.isnan(sc), mn, sc)

            cmax = jnp.max(sc, axis=1, keepdims=True)
            iota = jax.lax.broadcasted_iota(jnp.int32, sc.shape, 1)
            carg = jnp.min(
                jnp.where(sc == cmax, iota, K), axis=1, keepdims=True
            ) + k0

            better = cmax > run_max
            return (
                jnp.where(better, cmax, run_max),
                jnp.where(better, carg, run_idx),
            )

        init = (
            jnp.full((MBLK, 1), -jnp.inf, dtype=jnp.float32),
            jnp.zeros((MBLK, 1), dtype=jnp.int32),
        )
        _, idx = jax.lax.fori_loop(0, nchunks, body, init)
        out_ref[...] = idx.reshape(1, 1, MBLK)


def kernel(x_real, x_imag, cb_real, cb_imag):
    xr = x_real.reshape(M, D)
    xi = x_imag.reshape(M, D)

    nm = M // MBLK
    out = pl.pallas_call(
        _vq_kernel,
        grid=(nm,),
        in_specs=[
            pl.BlockSpec((MBLK, D), lambda m: (m, 0)),
            pl.BlockSpec((MBLK, D), lambda m: (m, 0)),
            pl.BlockSpec((K, D), lambda m: (0, 0)),
            pl.BlockSpec((K, D), lambda m: (0, 0)),
        ],
        out_specs=pl.BlockSpec((1, 1, MBLK), lambda m: (m, 0, 0)),
        out_shape=jax.ShapeDtypeStruct((nm, 1, MBLK), jnp.int32),
        scratch_shapes=[
            pltpu.VMEM((K, D), jnp.float32),
            pltpu.VMEM((K, D), jnp.float32),
        ],
    )(xr, xi, cb_real, cb_imag)
    return out.reshape(B, N)


# speedup vs baseline: 1.9878x; 1.0717x over previous
import functools

import jax
import jax.numpy as jnp
from jax.experimental import pallas as pl
from jax.experimental.pallas import tpu as pltpu

B, N, D = 16, 576, 256
K = 8192
M = B * N

MBLK = 256
KBLK = 4096
GUARD = 1e-5

_dot = functools.partial(
    jax.lax.dot_general,
    dimension_numbers=(((1,), (1,)), ((), ())),
    preferred_element_type=jnp.float32,
)


def _vq_kernel(xr_ref, xi_ref, cr_ref, ci_ref, out_ref, cpc_ref, cmc_ref):
    @pl.when(pl.program_id(0) == 0)
    def _prep():
        cr_all = cr_ref[...]
        ci_all = ci_ref[...]
        cpc_ref[...] = cr_all + ci_all
        cmc_ref[...] = ci_all - cr_all

    xr = xr_ref[...]
    xi = xi_ref[...]
    xd = xr - xi

    nchunks = K // KBLK
    nslices = KBLK // 128

    def re_im(k0):
        cr = cr_ref[pl.ds(k0, KBLK), :]
        cpc = cpc_ref[pl.ds(k0, KBLK), :]
        cmc = cmc_ref[pl.ds(k0, KBLK), :]
        p1 = _dot(xd, cr)
        p2 = _dot(xi, cpc)
        p3 = _dot(xr, cmc)
        return p1 + p2, p1 + p3

    def body_a(c, carry):
        m1, m2, lslot = carry
        k0 = c * KBLK
        re, im = re_im(k0)
        u = re * re + im * im
        for j in range(nslices):
            s = u[:, j * 128:(j + 1) * 128]
            slot = c * nslices + j
            gt = s > m1
            m2 = jnp.maximum(m2, jnp.minimum(m1, s))
            m1 = jnp.maximum(m1, s)
            lslot = jnp.where(gt, slot, lslot)
        return m1, m2, lslot

    carry = (
        jnp.full((MBLK, 128), -jnp.inf, jnp.float32),
        jnp.full((MBLK, 128), -jnp.inf, jnp.float32),
        jnp.zeros((MBLK, 128), jnp.int32),
    )
    for c in range(nchunks):
        carry = body_a(c, carry)
    m1, m2, lslot = carry

    umax = jnp.max(m1, axis=1, keepdims=True)
    thresh = umax * (1.0 - GUARD)
    cnt = jnp.sum((m1 >= thresh).astype(jnp.int32), axis=1, keepdims=True)
    cnt = cnt + jnp.sum(
        (m2 >= thresh).astype(jnp.int32), axis=1, keepdims=True
    )
    lane = jax.lax.broadcasted_iota(jnp.int32, (MBLK, 128), 1)
    kglob = lslot * 128 + lane
    argf = jnp.min(jnp.where(m1 == umax, kglob, K), axis=1, keepdims=True)
    out_ref[...] = argf.reshape(1, 1, MBLK)

    ambiguous = jnp.sum((cnt != 1).astype(jnp.int32)) > 0

    @pl.when(ambiguous)
    def _exact_path():
        def body(c, carry):
            run_max, run_idx = carry
            k0 = c * KBLK
            re, im = re_im(k0)
            are = jnp.abs(re)
            aim = jnp.abs(im)
            mx = jnp.maximum(are, aim)
            mn = jnp.minimum(are, aim)
            q = mn / mx
            sc = mx * jnp.sqrt(1.0 + q * q)
            sc = jnp.where(jnp.isnan(sc), mn, sc)

            cmax = jnp.max(sc, axis=1, keepdims=True)
            iota = jax.lax.broadcasted_iota(jnp.int32, sc.shape, 1)
            carg = jnp.min(
                jnp.where(sc == cmax, iota, K), axis=1, keepdims=True
            ) + k0

            better = cmax > run_max
            return (
                jnp.where(better, cmax, run_max),
                jnp.where(better, carg, run_idx),
            )

        init = (
            jnp.full((MBLK, 1), -jnp.inf, dtype=jnp.float32),
            jnp.zeros((MBLK, 1), dtype=jnp.int32),
        )
        _, idx = jax.lax.fori_loop(0, nchunks, body, init)
        out_ref[...] = idx.reshape(1, 1, MBLK)


def kernel(x_real, x_imag, cb_real, cb_imag):
    xr = x_real.reshape(M, D)
    xi = x_imag.reshape(M, D)

    nm = M // MBLK
    out = pl.pallas_call(
        _vq_kernel,
        grid=(nm,),
        in_specs=[
            pl.BlockSpec((MBLK, D), lambda m: (m, 0)),
            pl.BlockSpec((MBLK, D), lambda m: (m, 0)),
            pl.BlockSpec((K, D), lambda m: (0, 0)),
            pl.BlockSpec((K, D), lambda m: (0, 0)),
        ],
        out_specs=pl.BlockSpec((1, 1, MBLK), lambda m: (m, 0, 0)),
        out_shape=jax.ShapeDtypeStruct((nm, 1, MBLK), jnp.int32),
        scratch_shapes=[
            pltpu.VMEM((K, D), jnp.float32),
            pltpu.VMEM((K, D), jnp.float32),
        ],
    )(xr, xi, cb_real, cb_imag)
    return out.reshape(B, N)
